# baseline (speedup 1.0000x reference)
import functools

import jax
import jax.numpy as jnp
from jax import lax
from jax.experimental import pallas as pl
from jax.experimental.pallas import tpu as pltpu
from jax.experimental.pallas import tpu_sc as plsc

BATCH = 128
T_LEN = 32768
NCH = 4
OUT_LEN = 8192
MAX_T = T_LEN - OUT_LEN - 1

LANES = 128
NW = 32
BPW = BATCH // NW
CHUNK = 2048
NCHUNK = T_LEN // CHUNK
QW = 2048
WIN_ROWS = QW + 2


def _iota16():
    return lax.iota(jnp.int32, 16)


def _sums_body(x_hbm, out_hbm, buf0, buf1, stage, sem0, sem1):
    wid = lax.axis_index("s") * 2 + lax.axis_index("c")
    iota = _iota16()
    q4 = iota // 4
    r4 = iota % 4
    bufs = (buf0, buf1)
    sems = (sem0, sem1)

    def dma(i):
        k, ch = divmod(i, NCHUNK)
        b = BPW * wid + k
        return pltpu.async_copy(
            x_hbm.at[b, pl.ds(ch * CHUNK, CHUNK), :], bufs[i % 2],
            sems[i % 2])

    cur = dma(0)
    acc = jnp.zeros((16,), jnp.float32)
    for i in range(BPW * NCHUNK):
        k, ch = divmod(i, NCHUNK)
        nxt = dma(i + 1) if i + 1 < BPW * NCHUNK else None
        cur.wait()
        buf = bufs[i % 2]

        def inner(j, a):
            for u in range(16):
                rows = q4 + (j * 64 + u * 4)
                a = a + plsc.load_gather(buf, [rows, r4])
            return a

        acc = lax.fori_loop(0, CHUNK * NCH // 256, inner, acc)
        if ch == NCHUNK - 1:
            stage[pl.ds(16 * k, 16)] = acc
            acc = jnp.zeros((16,), jnp.float32)
        cur = nxt
    pltpu.sync_copy(stage, out_hbm.at[pl.ds(16 * BPW * wid, 16 * BPW)])


def _sums_sc(x):
    mesh = plsc.VectorSubcoreMesh(core_axis_name="c", subcore_axis_name="s")
    k = functools.partial(
        pl.kernel,
        mesh=mesh,
        out_type=jax.ShapeDtypeStruct((BATCH * 16,), jnp.float32),
        scratch_types=[
            pltpu.VMEM((CHUNK, NCH), jnp.float32),
            pltpu.VMEM((CHUNK, NCH), jnp.float32),
            pltpu.VMEM((16 * BPW,), jnp.float32),
            pltpu.SemaphoreType.DMA,
            pltpu.SemaphoreType.DMA,
        ],
        compiler_params=pltpu.CompilerParams(
            needs_layout_passes=False, use_tc_tiling_on_sc=False),
    )(_sums_body)
    return k(x)


def _bf16_rtne(x):
    u = lax.bitcast_convert_type(x, jnp.uint32)
    r = u + jnp.uint32(0x7FFF) + ((u >> jnp.uint32(16)) & jnp.uint32(1))
    return lax.bitcast_convert_type(r & jnp.uint32(0xFFFF0000), jnp.float32)


def _locnet_body(p_ref, w_ref, b_ref, o_ref):
    t = p_ref[...]
    t = t + pltpu.roll(t, LANES - 4, axis=1)
    t = t + pltpu.roll(t, LANES - 8, axis=1)
    m = t * jnp.float32(1.0 / T_LEN)
    mb = _bf16_rtne(m)
    wb = _bf16_rtne(w_ref[...])
    y = mb[:, 0:1] * wb[0:1, :]
    for c in range(1, NCH):
        y = y + mb[:, c:c + 1] * wb[c:c + 1, :]
    y = jax.nn.sigmoid(y + b_ref[0:1, :])
    st = jnp.floor(y * jnp.float32(T_LEN - 1))
    st = jnp.clip(st, 0.0, jnp.float32(MAX_T))
    o_ref[...] = st.astype(jnp.int32)


def _locnet_tc(partials, wp, brow):
    return pl.pallas_call(
        _locnet_body,
        in_specs=[
            pl.BlockSpec((BATCH, LANES), lambda: (0, 0)),
            pl.BlockSpec((8, LANES), lambda: (0, 0)),
            pl.BlockSpec((8, LANES), lambda: (0, 0)),
        ],
        out_specs=pl.BlockSpec((BATCH, LANES), lambda: (0, 0)),
        out_shape=jax.ShapeDtypeStruct((BATCH, LANES), jnp.int32),
    )(partials, wp, brow)


def _gather_body(x_hbm, st_hbm, out_hbm, st_v, win0, win1, oq,
                 sem0, sem1):
    wid = lax.axis_index("s") * 2 + lax.axis_index("c")
    pltpu.sync_copy(st_hbm.at[pl.ds(wid * 16, 16)], st_v)
    sv = st_v[...]
    iota = _iota16()

    wins = (win0, win1)
    sems = (sem0, sem1)

    nsteps = BPW * 16

    def lane(idx):
        g = plsc.load_gather(st_v, [jnp.broadcast_to(idx, (16,))])
        return g[0]

    def decode(i):
        k, q, c = i // 16, (i // 4) % 4, i % 4
        b = BPW * wid + k
        s = lane(NCH * k + c)
        se = pl.multiple_of(s - (s & 1), 2)
        dd = s - se
        return b, q, c, se, dd

    def issue(i, win, sem):
        b, q, c, se, dd = decode(i)
        pltpu.async_copy(
            x_hbm.at[b, pl.ds(se + q * QW, WIN_ROWS), :], win, sem)

    def wait_win(win, sem):
        pltpu.make_async_copy(
            x_hbm.at[0, pl.ds(0, WIN_ROWS), :], win, sem).wait()

    def extract(buf_ref, dd, c):
        cols = jnp.full((16,), 1, jnp.int32) * c

        def outer(jj, carry):
            for u in range(16):
                t0 = (jj * 16 + u) * 16
                v = plsc.load_gather(buf_ref, [iota + (t0 + dd), cols])
                plsc.store_scatter(oq, [iota + t0, cols], v)
            return carry

        lax.fori_loop(0, QW // 256, outer, 0)

    def step(i, win, sem):
        wait_win(win, sem)
        b, q, c, se, dd = decode(i)
        extract(win, dd, c)

        @pl.when(c == NCH - 1)
        def _():
            pltpu.sync_copy(oq, out_hbm.at[b, pl.ds(q * QW, QW), :])

        @pl.when(i + 2 < nsteps)
        def _():
            issue(i + 2, win, sem)

    issue(0, win0, sem0)
    issue(1, win1, sem1)

    def pair(j, carry):
        step(2 * j, win0, sem0)
        step(2 * j + 1, win1, sem1)
        return carry

    lax.fori_loop(0, nsteps // 2, pair, 0)


def _gather_sc(x, st4):
    mesh = plsc.VectorSubcoreMesh(core_axis_name="c", subcore_axis_name="s")
    k = functools.partial(
        pl.kernel,
        mesh=mesh,
        out_type=jax.ShapeDtypeStruct((BATCH, OUT_LEN, NCH), jnp.float32),
        scratch_types=[
            pltpu.VMEM((16,), jnp.int32),
            pltpu.VMEM((WIN_ROWS, NCH), jnp.float32),
            pltpu.VMEM((WIN_ROWS, NCH), jnp.float32),
            pltpu.VMEM((QW, NCH), jnp.float32),
            pltpu.SemaphoreType.DMA,
            pltpu.SemaphoreType.DMA,
        ],
        compiler_params=pltpu.CompilerParams(
            needs_layout_passes=False, use_tc_tiling_on_sc=False),
    )(_gather_body)
    return k(x, st4)


def kernel(X, W, b):
    batch, t_len, nch = X.shape
    assert (batch, t_len, nch) == (BATCH, T_LEN, NCH)
    partials = _sums_sc(X)
    ppad = jnp.pad(partials.reshape(BATCH, 16),
                   ((0, 0), (0, LANES - 16)))
    wp = jnp.pad(W, ((0, 8 - NCH), (0, LANES - NCH)))
    brow = jnp.broadcast_to(
        jnp.pad(b, (0, LANES - NCH)).reshape(1, LANES), (8, LANES))
    starts = _locnet_tc(ppad, wp, brow)
    st4 = starts[:, :NCH].reshape(-1)
    return _gather_sc(X, st4)


# speedup vs baseline: 66.2786x; 66.2786x over previous
import functools

import jax
import jax.numpy as jnp
from jax import lax
from jax.experimental import pallas as pl
from jax.experimental.pallas import tpu as pltpu
from jax.experimental.pallas import tpu_sc as plsc

BATCH = 128
T_LEN = 32768
NCH = 4
OUT_LEN = 8192
MAX_T = T_LEN - OUT_LEN - 1

LANES = 128
TB = T_LEN // LANES
NW = 32
BPW = BATCH // NW
CTB = 32
NCHUNK = TB // CTB
QW = OUT_LEN // 4
QTB = QW // LANES
WIN_TB = QTB + 1


def _iota16():
    return lax.iota(jnp.int32, 16)


def _sums_body(x_hbm, out_hbm, buf0, buf1, stage, sem0, sem1):
    wid = lax.axis_index("s") * 2 + lax.axis_index("c")
    bufs = (buf0, buf1)
    sems = (sem0, sem1)

    def dma(i):
        k, ci = divmod(i, NCHUNK)
        b = BPW * wid + k
        return pltpu.async_copy(
            x_hbm.at[b, pl.ds(ci * CTB, CTB), :, :], bufs[i % 2],
            sems[i % 2])

    cur = dma(0)
    accs = [jnp.zeros((16,), jnp.float32) for _ in range(NCH)]
    for i in range(BPW * NCHUNK):
        k, ci = divmod(i, NCHUNK)
        nxt = dma(i + 1) if i + 1 < BPW * NCHUNK else None
        cur.wait()
        buf = bufs[i % 2]

        def inner(tt, a):
            out = []
            for c in range(NCH):
                ac = a[c]
                for u in range(LANES // 16):
                    ac = ac + buf[tt, c, pl.ds(16 * u, 16)]
                out.append(ac)
            return tuple(out)

        accs = list(lax.fori_loop(0, CTB, inner, tuple(accs)))
        if ci == NCHUNK - 1:
            for c in range(NCH):
                stage[pl.ds(64 * k + 16 * c, 16)] = accs[c]
                accs[c] = jnp.zeros((16,), jnp.float32)
        cur = nxt
    pltpu.sync_copy(stage, out_hbm.at[pl.ds(64 * BPW * wid, 64 * BPW)])


def _sums_sc(x4):
    mesh = plsc.VectorSubcoreMesh(core_axis_name="c", subcore_axis_name="s")
    k = functools.partial(
        pl.kernel,
        mesh=mesh,
        out_type=jax.ShapeDtypeStruct((BATCH * 64,), jnp.float32),
        scratch_types=[
            pltpu.VMEM((CTB, NCH, LANES), jnp.float32),
            pltpu.VMEM((CTB, NCH, LANES), jnp.float32),
            pltpu.VMEM((64 * BPW,), jnp.float32),
            pltpu.SemaphoreType.DMA,
            pltpu.SemaphoreType.DMA,
        ],
        compiler_params=pltpu.CompilerParams(
            needs_layout_passes=False, use_tc_tiling_on_sc=False),
    )(_sums_body)
    return k(x4)


def _bf16_rtne(x):
    u = lax.bitcast_convert_type(x, jnp.uint32)
    r = u + jnp.uint32(0x7FFF) + ((u >> jnp.uint32(16)) & jnp.uint32(1))
    return lax.bitcast_convert_type(r & jnp.uint32(0xFFFF0000), jnp.float32)


def _locnet_body(p_ref, w_ref, b_ref, o_ref):
    t = p_ref[...]
    for sh in (8, 4, 2, 1):
        t = t + pltpu.roll(t, LANES - sh, axis=1)
    m = t * jnp.float32(1.0 / T_LEN)
    mb = _bf16_rtne(m)
    wb = _bf16_rtne(w_ref[...])
    y = mb[:, 0:1] * wb[0:1, :]
    for c in range(1, NCH):
        y = y + mb[:, 16 * c:16 * c + 1] * wb[c:c + 1, :]
    y = jax.nn.sigmoid(y + b_ref[0:1, :])
    st = jnp.floor(y * jnp.float32(T_LEN - 1))
    st = jnp.clip(st, 0.0, jnp.float32(MAX_T))
    o_ref[...] = st.astype(jnp.int32)


def _locnet_tc(partials, wp, brow):
    return pl.pallas_call(
        _locnet_body,
        in_specs=[
            pl.BlockSpec((BATCH, LANES), lambda: (0, 0)),
            pl.BlockSpec((8, LANES), lambda: (0, 0)),
            pl.BlockSpec((8, LANES), lambda: (0, 0)),
        ],
        out_specs=pl.BlockSpec((BATCH, LANES), lambda: (0, 0)),
        out_shape=jax.ShapeDtypeStruct((BATCH, LANES), jnp.int32),
    )(partials, wp, brow)


def _gather_body(x_hbm, st_hbm, out_hbm, st_v, win0, win1, oq,
                 sem0, sem1):
    wid = lax.axis_index("s") * 2 + lax.axis_index("c")
    pltpu.sync_copy(st_hbm.at[pl.ds(wid * 16, 16)], st_v)
    iota = _iota16()

    nsteps = BPW * 16

    def lane(idx):
        g = plsc.load_gather(st_v, [jnp.broadcast_to(idx, (16,))])
        return g[0]

    def decode(i):
        k, q, c = i // 16, (i // 4) % 4, i % 4
        b = BPW * wid + k
        s = lane(NCH * k + c)
        t0 = s + q * QW
        tt0 = t0 // LANES
        r = t0 % LANES
        return b, q, c, tt0, r

    def issue(i, win, sem):
        b, q, c, tt0, r = decode(i)
        pltpu.async_copy(
            x_hbm.at[b, pl.ds(tt0, WIN_TB), c, :], win, sem)

    def wait_win(win, sem):
        pltpu.make_async_copy(
            x_hbm.at[0, pl.ds(0, WIN_TB), 0, :], win, sem).wait()

    def extract(buf_ref, r):
        def outer(jj, carry):
            for u16 in range(16):
                u = (jj * 16 + u16) * 16 + iota
                f = u + r
                v = plsc.load_gather(buf_ref, [f >> 7, f & 127])
                plsc.store_scatter(oq, [u >> 7, u & 127], v)
            return carry

        lax.fori_loop(0, QW // 256, outer, 0)

    def step(i, win, sem):
        wait_win(win, sem)
        b, q, c, tt0, r = decode(i)
        extract(win, r)
        pltpu.sync_copy(oq, out_hbm.at[b, pl.ds(q * QTB, QTB), c, :])

        @pl.when(i + 2 < nsteps)
        def _():
            issue(i + 2, win, sem)

    issue(0, win0, sem0)
    issue(1, win1, sem1)

    def pair(j, carry):
        step(2 * j, win0, sem0)
        step(2 * j + 1, win1, sem1)
        return carry

    lax.fori_loop(0, nsteps // 2, pair, 0)


def _gather_sc(x4, st4):
    mesh = plsc.VectorSubcoreMesh(core_axis_name="c", subcore_axis_name="s")
    k = functools.partial(
        pl.kernel,
        mesh=mesh,
        out_type=jax.ShapeDtypeStruct((BATCH, OUT_LEN // LANES, NCH, LANES),
                                      jnp.float32),
        scratch_types=[
            pltpu.VMEM((16,), jnp.int32),
            pltpu.VMEM((WIN_TB, LANES), jnp.float32),
            pltpu.VMEM((WIN_TB, LANES), jnp.float32),
            pltpu.VMEM((QTB, LANES), jnp.float32),
            pltpu.SemaphoreType.DMA,
            pltpu.SemaphoreType.DMA,
        ],
        compiler_params=pltpu.CompilerParams(
            needs_layout_passes=False, use_tc_tiling_on_sc=False),
    )(_gather_body)
    return k(x4, st4)


def kernel(X, W, b):
    batch, t_len, nch = X.shape
    assert (batch, t_len, nch) == (BATCH, T_LEN, NCH)
    x4 = X.transpose(0, 2, 1).reshape(BATCH, NCH, TB, LANES)
    x4 = x4.transpose(0, 2, 1, 3)
    partials = _sums_sc(x4)
    ppad = jnp.pad(partials.reshape(BATCH, 64),
                   ((0, 0), (0, LANES - 64)))
    wp = jnp.pad(W, ((0, 8 - NCH), (0, LANES - NCH)))
    brow = jnp.broadcast_to(
        jnp.pad(b, (0, LANES - NCH)).reshape(1, LANES), (8, LANES))
    starts = _locnet_tc(ppad, wp, brow)
    st4 = starts[:, :NCH].reshape(-1)
    o4 = _gather_sc(x4, st4)
    out = o4.transpose(0, 2, 1, 3).reshape(BATCH, NCH, OUT_LEN)
    return out.transpose(0, 2, 1)


# speedup vs baseline: 78.7305x; 1.1879x over previous
import functools

import jax
import jax.numpy as jnp
from jax import lax
from jax.experimental import pallas as pl
from jax.experimental.pallas import tpu as pltpu
from jax.experimental.pallas import tpu_sc as plsc

BATCH = 128
T_LEN = 32768
NCH = 4
OUT_LEN = 8192
MAX_T = T_LEN - OUT_LEN - 1

LANES = 128
TB = T_LEN // LANES
NW = 32
BPW = BATCH // NW
CTB = 64
NCHUNK = TB // CTB
QTB = OUT_LEN // LANES
WIN_TB = QTB + 1


def _iota16():
    return lax.iota(jnp.int32, 16)


def _sums_body(x_hbm, out_hbm, buf0, buf1, stage, sem0, sem1):
    wid = lax.axis_index("s") * 2 + lax.axis_index("c")
    bufs = (buf0, buf1)
    sems = (sem0, sem1)

    def dma(i):
        k, ci = divmod(i, NCHUNK)
        b = BPW * wid + k
        return pltpu.async_copy(
            x_hbm.at[b, pl.ds(ci * CTB, CTB), :, :], bufs[i % 2],
            sems[i % 2])

    cur = dma(0)
    accs = [jnp.zeros((16,), jnp.float32) for _ in range(NCH)]
    for i in range(BPW * NCHUNK):
        k, ci = divmod(i, NCHUNK)
        nxt = dma(i + 1) if i + 1 < BPW * NCHUNK else None
        cur.wait()
        buf = bufs[i % 2]

        def inner(tt, a):
            out = []
            for c in range(NCH):
                ac = a[c]
                for u in range(LANES // 16):
                    ac = ac + buf[tt, c, pl.ds(16 * u, 16)]
                out.append(ac)
            return tuple(out)

        accs = list(lax.fori_loop(0, CTB, inner, tuple(accs)))
        if ci == NCHUNK - 1:
            for c in range(NCH):
                stage[pl.ds(64 * k + 16 * c, 16)] = accs[c]
                accs[c] = jnp.zeros((16,), jnp.float32)
        cur = nxt
    pltpu.sync_copy(stage, out_hbm.at[pl.ds(64 * BPW * wid, 64 * BPW)])


def _sums_sc(x4):
    mesh = plsc.VectorSubcoreMesh(core_axis_name="c", subcore_axis_name="s")
    k = functools.partial(
        pl.kernel,
        mesh=mesh,
        out_type=jax.ShapeDtypeStruct((BATCH * 64,), jnp.float32),
        scratch_types=[
            pltpu.VMEM((CTB, NCH, LANES), jnp.float32),
            pltpu.VMEM((CTB, NCH, LANES), jnp.float32),
            pltpu.VMEM((64 * BPW,), jnp.float32),
            pltpu.SemaphoreType.DMA,
            pltpu.SemaphoreType.DMA,
        ],
        compiler_params=pltpu.CompilerParams(
            needs_layout_passes=False, use_tc_tiling_on_sc=False),
    )(_sums_body)
    return k(x4)


def _bf16_rtne(x):
    u = lax.bitcast_convert_type(x, jnp.uint32)
    r = u + jnp.uint32(0x7FFF) + ((u >> jnp.uint32(16)) & jnp.uint32(1))
    return lax.bitcast_convert_type(r & jnp.uint32(0xFFFF0000), jnp.float32)


def _locnet_body(p_ref, w_ref, b_ref, o_ref):
    t = p_ref[...]
    for sh in (8, 4, 2, 1):
        t = t + pltpu.roll(t, LANES - sh, axis=1)
    m = t * jnp.float32(1.0 / T_LEN)
    mb = _bf16_rtne(m)
    wb = _bf16_rtne(w_ref[...])
    y = mb[:, 0:1] * wb[0:1, :]
    for c in range(1, NCH):
        y = y + mb[:, 16 * c:16 * c + 1] * wb[c:c + 1, :]
    y = jax.nn.sigmoid(y + b_ref[0:1, :])
    st = jnp.floor(y * jnp.float32(T_LEN - 1))
    st = jnp.clip(st, 0.0, jnp.float32(MAX_T))
    o_ref[...] = st.astype(jnp.int32)


def _locnet_tc(partials, wp, brow):
    return pl.pallas_call(
        _locnet_body,
        in_specs=[
            pl.BlockSpec((BATCH, LANES), lambda: (0, 0)),
            pl.BlockSpec((8, LANES), lambda: (0, 0)),
            pl.BlockSpec((8, LANES), lambda: (0, 0)),
        ],
        out_specs=pl.BlockSpec((BATCH, LANES), lambda: (0, 0)),
        out_shape=jax.ShapeDtypeStruct((BATCH, LANES), jnp.int32),
    )(partials, wp, brow)


def _gather_body(x_hbm, st_hbm, out_hbm, st_v, win0, win1, oq0, oq1,
                 sem0, sem1, osem0, osem1):
    wid = lax.axis_index("s") * 2 + lax.axis_index("c")
    pltpu.sync_copy(st_hbm.at[pl.ds(wid * 16, 16)], st_v)
    iota = _iota16()

    nsteps = BPW * NCH

    def lane(idx):
        g = plsc.load_gather(st_v, [jnp.broadcast_to(idx, (16,))])
        return g[0]

    def decode(i):
        k, c = i // NCH, i % NCH
        b = BPW * wid + k
        s = lane(NCH * k + c)
        tt0 = s // LANES
        r = s % LANES
        return b, c, tt0, r

    def issue(i, win, sem):
        b, c, tt0, r = decode(i)
        pltpu.async_copy(
            x_hbm.at[b, pl.ds(tt0, WIN_TB), c, :], win, sem)

    def wait_win(win, sem):
        pltpu.make_async_copy(
            x_hbm.at[0, pl.ds(0, WIN_TB), 0, :], win, sem).wait()

    def drain_out(oq, osem):
        pltpu.make_async_copy(
            x_hbm.at[0, pl.ds(0, QTB), 0, :], oq, osem).wait()

    def extract(buf_ref, oq, r):
        def outer(jj, carry):
            for u16 in range(16):
                u = (jj * 16 + u16) * 16 + iota
                f = u + r
                v = plsc.load_gather(buf_ref, [f >> 7, f & 127])
                plsc.store_scatter(oq, [u >> 7, u & 127], v)
            return carry

        lax.fori_loop(0, OUT_LEN // 256, outer, 0)

    def step(i, win, sem, oq, osem, drain):
        wait_win(win, sem)
        b, c, tt0, r = decode(i)
        if drain:
            drain_out(oq, osem)
        extract(win, oq, r)
        pltpu.async_copy(oq, out_hbm.at[b, pl.ds(0, QTB), c, :], osem)

        @pl.when(i + 2 < nsteps)
        def _():
            issue(i + 2, win, sem)

    issue(0, win0, sem0)
    issue(1, win1, sem1)
    step(0, win0, sem0, oq0, osem0, False)
    step(1, win1, sem1, oq1, osem1, False)

    def pair(j, carry):
        step(2 * j, win0, sem0, oq0, osem0, True)
        step(2 * j + 1, win1, sem1, oq1, osem1, True)
        return carry

    lax.fori_loop(1, nsteps // 2, pair, 0)
    drain_out(oq0, osem0)
    drain_out(oq1, osem1)


def _gather_sc(x4, st4):
    mesh = plsc.VectorSubcoreMesh(core_axis_name="c", subcore_axis_name="s")
    k = functools.partial(
        pl.kernel,
        mesh=mesh,
        out_type=jax.ShapeDtypeStruct((BATCH, OUT_LEN // LANES, NCH, LANES),
                                      jnp.float32),
        scratch_types=[
            pltpu.VMEM((16,), jnp.int32),
            pltpu.VMEM((WIN_TB, LANES), jnp.float32),
            pltpu.VMEM((WIN_TB, LANES), jnp.float32),
            pltpu.VMEM((QTB, LANES), jnp.float32),
            pltpu.VMEM((QTB, LANES), jnp.float32),
            pltpu.SemaphoreType.DMA,
            pltpu.SemaphoreType.DMA,
            pltpu.SemaphoreType.DMA,
            pltpu.SemaphoreType.DMA,
        ],
        compiler_params=pltpu.CompilerParams(
            needs_layout_passes=False, use_tc_tiling_on_sc=False),
    )(_gather_body)
    return k(x4, st4)


def kernel(X, W, b):
    batch, t_len, nch = X.shape
    assert (batch, t_len, nch) == (BATCH, T_LEN, NCH)
    x4 = X.transpose(0, 2, 1).reshape(BATCH, NCH, TB, LANES)
    x4 = x4.transpose(0, 2, 1, 3)
    partials = _sums_sc(x4)
    ppad = jnp.pad(partials.reshape(BATCH, 64),
                   ((0, 0), (0, LANES - 64)))
    wp = jnp.pad(W, ((0, 8 - NCH), (0, LANES - NCH)))
    brow = jnp.broadcast_to(
        jnp.pad(b, (0, LANES - NCH)).reshape(1, LANES), (8, LANES))
    starts = _locnet_tc(ppad, wp, brow)
    st4 = starts[:, :NCH].reshape(-1)
    o4 = _gather_sc(x4, st4)
    out = o4.transpose(0, 2, 1, 3).reshape(BATCH, NCH, OUT_LEN)
    return out.transpose(0, 2, 1)
